# baseline (speedup 1.0000x reference)
import jax
import jax.numpy as jnp
from jax import lax
from jax.experimental import pallas as pl
from jax.experimental.pallas import tpu as pltpu
from jax.experimental.pallas import tpu_sc as plsc

B, NF, E, U, NNZ = 4, 512, 8192, 16384, 131072
NC, NS, L = 2, 16, 16
Q = NF // 128
EPT = NNZ // NS
CH = 2048
P = (B * U) // (CH * NC)
RPT = CH // NS
SEL = EPT + L


def _body(feat_ref, bidx_ref, ridx_ref, cidx_ref, gval_ref, occ_ref, out_ref,
          key_vm, fidx_vm, vals_vm, sel_dest, sel_fidx, sel_vals,
          gstage, dstage, rowbuf, shared, gsem):
    cid = lax.axis_index("c")
    sid = lax.axis_index("s")
    base = sid * EPT
    i16 = jnp.arange(L, dtype=jnp.int32)
    zf = jnp.zeros((L,), jnp.float32)
    zi = jnp.zeros((L,), jnp.int32)

    pltpu.sync_copy(bidx_ref.at[pl.ds(base, EPT)], key_vm)
    pltpu.sync_copy(cidx_ref.at[pl.ds(base, EPT)], sel_dest.at[pl.ds(0, EPT)])
    pltpu.sync_copy(ridx_ref.at[pl.ds(base, EPT)], fidx_vm)
    pltpu.sync_copy(gval_ref.at[pl.ds(base, EPT)], vals_vm)

    def init_body(i, _):
        s = pl.ds(i * L, L)
        b = key_vm[s]
        key_vm[s] = b * U + sel_dest[s]
        fidx_vm[s] = b * E + fidx_vm[s]
        return 0
    lax.fori_loop(0, EPT // L, init_body, 0)

    descs = [
        pltpu.async_copy(occ_ref.at[key_vm.at[pl.ds(i * 128, 128)]],
                         sel_vals.at[pl.ds(i * 128, 128)], gsem)
        for i in range(EPT // 128)
    ]
    for d in descs:
        d.wait()

    def div_body(i, _):
        s = pl.ds(i * L, L)
        vals_vm[s] = vals_vm[s] / sel_vals[s]
        return 0
    lax.fori_loop(0, EPT // L, div_body, 0)

    def selz(i, _):
        s = pl.ds(i * L, L)
        sel_dest[s] = zi
        sel_fidx[s] = zi
        return 0
    lax.fori_loop(0, EPT // L, selz, 0)

    def selzv(i, _):
        sel_vals[pl.ds(i * L, L)] = zf
        return 0
    lax.fori_loop(0, SEL // L, selzv, 0)

    def pass_body(p, _):
        lo = (p * NC + cid) * CH
        myrow = sid * RPT

        for q in range(Q):
            def zb_row(j, _):
                def zb_col(k, _):
                    rowbuf[j, q, pl.ds(k * L, L)] = zf
                    return 0
                lax.fori_loop(0, 128 // L, zb_col, 0)
                return 0
            lax.fori_loop(0, L, zb_row, 0)

        def z_body(i, _):
            pltpu.sync_copy(rowbuf, shared.at[pl.ds(myrow + i * L, L)])
            return 0
        lax.fori_loop(0, RPT // L, z_body, 0)
        plsc.subcore_barrier()

        def f_body(i, off):
            s = pl.ds(i * L, L)
            key = key_vm[s]
            m = (key >= lo) & (key < lo + CH)
            mi = jnp.where(m, jnp.int32(1), jnp.int32(0))
            pref = plsc.cumsum(mi)
            pos = off + pref - 1
            plsc.store_scatter(sel_dest, [pos], key - lo, mask=m)
            plsc.store_scatter(sel_fidx, [pos], fidx_vm[s], mask=m)
            plsc.store_scatter(sel_vals, [pos], vals_vm[s], mask=m)
            return off + pref[L - 1]
        mcnt = lax.fori_loop(0, EPT // L, f_body, jnp.int32(0))

        sel_vals[pl.ds(mcnt, L)] = zf
        nblk = (mcnt + (L - 1)) // L

        def blk_body(blk, _):
            s = pl.ds(blk * L, L)
            gstage[...] = sel_fidx[s]
            pltpu.async_copy(feat_ref.at[gstage], rowbuf, gsem).wait()
            vvec = sel_vals[s]

            for q in range(Q):
                def sc_body(k, _):
                    cols = jnp.full((L,), k, jnp.int32)
                    qs = jnp.full((L,), q, jnp.int32)
                    colv = plsc.load_gather(rowbuf, [i16, qs, cols])
                    plsc.store_scatter(rowbuf, [i16, qs, cols], colv * vvec)
                    return 0
                lax.fori_loop(0, 128, sc_body, 0)

            dstage[...] = sel_dest[s]
            pltpu.sync_copy(rowbuf, shared.at[dstage], add=True)
            return 0
        lax.fori_loop(0, nblk, blk_body, 0)

        plsc.subcore_barrier()
        pltpu.sync_copy(shared.at[pl.ds(myrow, RPT)],
                        out_ref.at[pl.ds(lo + myrow, RPT)])
        return 0
    lax.fori_loop(0, P, pass_body, 0)


_sc = pl.kernel(
    _body,
    out_type=jax.ShapeDtypeStruct((B * U, Q, 128), jnp.float32),
    mesh=plsc.VectorSubcoreMesh(core_axis_name="c", subcore_axis_name="s"),
    compiler_params=pltpu.CompilerParams(needs_layout_passes=False),
    scratch_types=[
        pltpu.VMEM((EPT,), jnp.int32),
        pltpu.VMEM((EPT,), jnp.int32),
        pltpu.VMEM((EPT,), jnp.float32),
        pltpu.VMEM((EPT,), jnp.int32),
        pltpu.VMEM((EPT,), jnp.int32),
        pltpu.VMEM((SEL,), jnp.float32),
        pltpu.VMEM((L,), jnp.int32),
        pltpu.VMEM((L,), jnp.int32),
        pltpu.VMEM((L, Q, 128), jnp.float32),
        pltpu.VMEM_SHARED((CH, Q, 128), jnp.float32),
        pltpu.SemaphoreType.DMA,
    ],
)


@jax.jit
def kernel(features, batch_idx, row_idx, col_idx, group_values, occurrences):
    feat3d = features.transpose(0, 2, 1).reshape(B * E, Q, 128)
    occf = occurrences.reshape(B * U)
    out3d = _sc(feat3d, batch_idx, row_idx, col_idx, group_values, occf)
    return out3d.reshape(B, U, NF).transpose(0, 2, 1)
